# baseline (speedup 1.0000x reference)
import functools

import jax
import jax.numpy as jnp
from jax.experimental import pallas as pl
from jax.experimental.pallas import tpu as pltpu
from jax.experimental.pallas import tpu_sc as plsc

NUM_BUCKETS = 32
MAX_DISTANCE = 128
NUM_HEADS = 16
QLEN = 2048
KLEN = 2048
TAB = 4096


def _table_body(off_ref, emb_ref, tab_ref):
    off = off_ref[0]
    j = jax.lax.broadcasted_iota(jnp.int32, (1, TAB), 1)
    rel_pos = j - (QLEN - 1) + off
    nn = -rel_pos
    half = NUM_BUCKETS // 2
    ret_hi = jnp.where(nn < 0, half, 0)
    na = jnp.abs(nn)
    max_exact = half // 2
    is_small = na < max_exact
    eps = jnp.finfo(jnp.float32).eps
    val_if_large = max_exact + (
        jnp.log(na.astype(jnp.float32) / max_exact + eps)
        / jnp.log(MAX_DISTANCE / max_exact)
        * (half - max_exact)
    ).astype(jnp.int32)
    val_if_large = jnp.minimum(val_if_large, half - 1)
    bucket = ret_hi + jnp.where(is_small, na, val_if_large)
    rowid = jax.lax.broadcasted_iota(jnp.int32, (NUM_BUCKETS, TAB), 0)
    onehot = (bucket == rowid).astype(jnp.float32)
    tab_ref[...] = jnp.dot(
        emb_ref[...], onehot, preferred_element_type=jnp.float32
    )


def _build_table(off, emb):
    return pl.pallas_call(
        _table_body,
        out_shape=jax.ShapeDtypeStruct((NUM_HEADS, TAB), jnp.float32),
        in_specs=[
            pl.BlockSpec(memory_space=pltpu.SMEM),
            pl.BlockSpec(memory_space=pltpu.VMEM),
        ],
        out_specs=pl.BlockSpec(memory_space=pltpu.VMEM),
    )(off, emb)


_ROWS = NUM_HEADS * QLEN
_ROWS_PER_TEC = _ROWS // 32
_BLOCKS_PER_TEC = _ROWS_PER_TEC // 8


def _sc_expand(table):
    mesh = plsc.VectorSubcoreMesh(core_axis_name="c", subcore_axis_name="s")

    @functools.partial(
        pl.kernel,
        mesh=mesh,
        out_type=jax.ShapeDtypeStruct((_ROWS, KLEN), jnp.float32),
        scratch_types=[
            pltpu.VMEM((TAB,), jnp.float32),
            pltpu.VMEM((3, 16, KLEN), jnp.float32),
            pltpu.SemaphoreType.DMA,
        ],
    )
    def body(tab_hbm, out_hbm, tab_v, stg_v, sem):
        c = jax.lax.axis_index("c")
        s = jax.lax.axis_index("s")
        wid = c * 16 + s
        head = wid // 2
        pltpu.sync_copy(tab_hbm.at[head], tab_v)
        base = wid * _ROWS_PER_TEC
        q_base = jax.lax.rem(base, QLEN)

        def fill(b, q0):
            toffs = []
            for sub in range(16):
                o = (QLEN - 1) - (q0 + sub)
                toffs.append(o)

            @plsc.parallel_loop(0, KLEN // 16, unroll=2)
            def cp(t):
                for sub in range(16):
                    src = toffs[sub] + t * 16
                    stg_v[b, sub, pl.ds(t * 16, 16)] = tab_v[pl.ds(src, 16)]

        def drain():
            pltpu.make_async_copy(
                stg_v.at[0], out_hbm.at[pl.ds(0, 16), :], sem
            ).wait()

        nblk = _ROWS_PER_TEC // 16

        def chunk(g, carry):
            for b in range(3):
                blk = g * 3 + b

                @pl.when(blk < nblk)
                def _(b=b, blk=blk):
                    @pl.when(g > 0)
                    def _():
                        drain()

                    fill(b, q_base + blk * 16)
                    r0 = pl.multiple_of(base + blk * 16, 8)
                    pltpu.async_copy(
                        stg_v.at[b], out_hbm.at[pl.ds(r0, 16), :], sem
                    )
            return carry

        jax.lax.fori_loop(0, (nblk + 2) // 3, chunk, 0)
        drain()
        drain()
        drain()

    return body(table)


def kernel(qlen, klen, rel_embedding):
    emb = jnp.asarray(rel_embedding, jnp.float32)
    off = (jnp.asarray(klen, jnp.int32) - jnp.asarray(qlen, jnp.int32)).reshape(1)
    table = _build_table(off, emb)
    out = _sc_expand(table)
    return out.reshape(1, NUM_HEADS, QLEN, KLEN)


# speedup vs baseline: 1.0070x; 1.0070x over previous
import functools

import jax
import jax.numpy as jnp
from jax.experimental import pallas as pl
from jax.experimental.pallas import tpu as pltpu
from jax.experimental.pallas import tpu_sc as plsc

NUM_BUCKETS = 32
MAX_DISTANCE = 128
NUM_HEADS = 16
QLEN = 2048
KLEN = 2048
TAB = 4096


def _table_body(off_ref, emb_ref, tab_ref):
    off = off_ref[0]
    j = jax.lax.broadcasted_iota(jnp.int32, (1, TAB), 1)
    rel_pos = j - (QLEN - 1) + off
    nn = -rel_pos
    half = NUM_BUCKETS // 2
    ret_hi = jnp.where(nn < 0, half, 0)
    na = jnp.abs(nn)
    max_exact = half // 2
    is_small = na < max_exact
    eps = jnp.finfo(jnp.float32).eps
    val_if_large = max_exact + (
        jnp.log(na.astype(jnp.float32) / max_exact + eps)
        / jnp.log(MAX_DISTANCE / max_exact)
        * (half - max_exact)
    ).astype(jnp.int32)
    val_if_large = jnp.minimum(val_if_large, half - 1)
    bucket = ret_hi + jnp.where(is_small, na, val_if_large)
    rowid = jax.lax.broadcasted_iota(jnp.int32, (NUM_BUCKETS, TAB), 0)
    onehot = (bucket == rowid).astype(jnp.float32)
    tab_ref[...] = jnp.dot(
        emb_ref[...], onehot, preferred_element_type=jnp.float32
    )


def _build_table(off, emb):
    return pl.pallas_call(
        _table_body,
        out_shape=jax.ShapeDtypeStruct((NUM_HEADS, TAB), jnp.float32),
        in_specs=[
            pl.BlockSpec(memory_space=pltpu.SMEM),
            pl.BlockSpec(memory_space=pltpu.VMEM),
        ],
        out_specs=pl.BlockSpec(memory_space=pltpu.VMEM),
    )(off, emb)


_ROWS = NUM_HEADS * QLEN
_ROWS_PER_TEC = _ROWS // 32
_BLOCKS_PER_TEC = _ROWS_PER_TEC // 8


def _sc_expand(table):
    mesh = plsc.VectorSubcoreMesh(core_axis_name="c", subcore_axis_name="s")

    @functools.partial(
        pl.kernel,
        mesh=mesh,
        out_type=jax.ShapeDtypeStruct((_ROWS, KLEN), jnp.float32),
        scratch_types=[
            pltpu.VMEM((TAB,), jnp.float32),
            pltpu.VMEM((5, 8, KLEN), jnp.float32),
            pltpu.SemaphoreType.DMA,
        ],
    )
    def body(tab_hbm, out_hbm, tab_v, stg_v, sem):
        c = jax.lax.axis_index("c")
        s = jax.lax.axis_index("s")
        wid = c * 16 + s
        head = wid // 2
        pltpu.sync_copy(tab_hbm.at[head], tab_v)
        base = wid * _ROWS_PER_TEC
        q_base = jax.lax.rem(base, QLEN)

        def fill(b, q0):
            toffs = []
            for sub in range(8):
                o = (QLEN - 1) - (q0 + sub)
                toffs.append(o)

            @plsc.parallel_loop(0, KLEN // 16, unroll=4)
            def cp(t):
                for sub in range(8):
                    src = toffs[sub] + t * 16
                    stg_v[b, sub, pl.ds(t * 16, 16)] = tab_v[pl.ds(src, 16)]

        def drain():
            pltpu.make_async_copy(
                stg_v.at[0], out_hbm.at[pl.ds(0, 8), :], sem
            ).wait()

        def chunk(g, carry):
            for b in range(5):
                blk = g * 5 + b

                @pl.when(blk < _BLOCKS_PER_TEC)
                def _(b=b, blk=blk):
                    @pl.when(g > 0)
                    def _():
                        drain()

                    fill(b, q_base + blk * 8)
                    r0 = pl.multiple_of(base + blk * 8, 8)
                    pltpu.async_copy(
                        stg_v.at[b], out_hbm.at[pl.ds(r0, 8), :], sem
                    )
            return carry

        jax.lax.fori_loop(0, (_BLOCKS_PER_TEC + 4) // 5, chunk, 0)
        for _ in range(5):
            drain()

    return body(table)


def kernel(qlen, klen, rel_embedding):
    emb = jnp.asarray(rel_embedding, jnp.float32)
    off = (jnp.asarray(klen, jnp.int32) - jnp.asarray(qlen, jnp.int32)).reshape(1)
    table = _build_table(off, emb)
    out = _sc_expand(table)
    return out.reshape(1, NUM_HEADS, QLEN, KLEN)
